# baseline (speedup 1.0000x reference)
import functools

import jax
import jax.numpy as jnp
from jax.experimental import pallas as pl
from jax.experimental.pallas import tpu as pltpu

_N_TOKENS = 8192
_N_CODES = 8192
_DIM = 256
_TBLK = 1024
_CBLK = 1024


def _argmin_body(x_ref, cb_ref, t1_ref, t3_ref, idx_ref, minval, minidx):
    c = pl.program_id(1)
    ncb = pl.num_programs(1)
    mm = jax.lax.dot_general(
        x_ref[...], cb_ref[...],
        dimension_numbers=(((1,), (1,)), ((), ())),
        preferred_element_type=jnp.float32,
    )
    d = (t1_ref[...] - 2.0 * mm) + t3_ref[...]
    bmin = jnp.min(d, axis=1, keepdims=True)
    iota = jax.lax.broadcasted_iota(jnp.int32, d.shape, 1)
    big = jnp.int32(2**30)
    bidx = jnp.min(jnp.where(d == bmin, iota, big), axis=1, keepdims=True)
    bidx = bidx + c * _CBLK

    @pl.when(c == 0)
    def _():
        minval[...] = bmin
        minidx[...] = bidx

    @pl.when(c > 0)
    def _():
        better = bmin < minval[...]
        minval[...] = jnp.where(better, bmin, minval[...])
        minidx[...] = jnp.where(better, bidx, minidx[...])

    @pl.when(c == ncb - 1)
    def _():
        idx_ref[...] = minidx[...]


@functools.partial(jax.jit, static_argnums=())
def _stage_argmin(x, cb, t1, t3):
    grid = (_N_TOKENS // _TBLK, _N_CODES // _CBLK)
    return pl.pallas_call(
        _argmin_body,
        grid=grid,
        in_specs=[
            pl.BlockSpec((_TBLK, _DIM), lambda t, c: (t, 0)),
            pl.BlockSpec((_CBLK, _DIM), lambda t, c: (c, 0)),
            pl.BlockSpec((_TBLK, 1), lambda t, c: (t, 0)),
            pl.BlockSpec((1, _CBLK), lambda t, c: (0, c)),
        ],
        out_specs=pl.BlockSpec((_TBLK, 1), lambda t, c: (t, 0)),
        out_shape=jax.ShapeDtypeStruct((_N_TOKENS, 1), jnp.int32),
        scratch_shapes=[
            pltpu.VMEM((_TBLK, 1), jnp.float32),
            pltpu.VMEM((_TBLK, 1), jnp.int32),
        ],
    )(x, cb, t1, t3)


def kernel(input_data, codebooks):
    remainder = input_data
    final_quantized = jnp.zeros_like(input_data)
    used = []
    for i in range(codebooks.shape[0]):
        cb = codebooks[i]
        t1 = jnp.sum(remainder ** 2, axis=1, keepdims=True)
        t3 = jnp.sum(cb.T ** 2, axis=0, keepdims=True)
        min_idx = _stage_argmin(remainder, cb, t1, t3)[:, 0]
        q = cb[min_idx]
        remainder = remainder - q
        final_quantized = final_quantized + q
        used.append(
            jnp.zeros((_N_CODES,), jnp.int32).at[min_idx].set(1))
    codebooks_used = jnp.stack(used, axis=0)
    return final_quantized, codebooks_used, codebooks


# speedup vs baseline: 1.1713x; 1.1713x over previous
import functools

import jax
import jax.numpy as jnp
from jax.experimental import pallas as pl
from jax.experimental.pallas import tpu as pltpu

_N_TOKENS = 8192
_N_CODES = 8192
_DIM = 256
_TBLK = 2048
_CBLK = 2048


def _argmin_body(x_ref, cb_ref, t1_ref, t3_ref, idx_ref, minval, minidx):
    c = pl.program_id(1)
    ncb = pl.num_programs(1)
    mm = jax.lax.dot_general(
        x_ref[...], cb_ref[...],
        dimension_numbers=(((1,), (1,)), ((), ())),
        preferred_element_type=jnp.float32,
    )
    d = (t1_ref[...] - 2.0 * mm) + t3_ref[...]
    bmin = jnp.min(d, axis=1, keepdims=True)
    iota = jax.lax.broadcasted_iota(jnp.int32, d.shape, 1)
    big = jnp.int32(2**30)
    bidx = jnp.min(jnp.where(d == bmin, iota, big), axis=1, keepdims=True)
    bidx = bidx + c * _CBLK

    @pl.when(c == 0)
    def _():
        minval[...] = bmin
        minidx[...] = bidx

    @pl.when(c > 0)
    def _():
        better = bmin < minval[...]
        minval[...] = jnp.where(better, bmin, minval[...])
        minidx[...] = jnp.where(better, bidx, minidx[...])

    @pl.when(c == ncb - 1)
    def _():
        idx_ref[...] = minidx[...]


@functools.partial(jax.jit, static_argnums=())
def _stage_argmin(x, cb, t1, t3):
    grid = (_N_TOKENS // _TBLK, _N_CODES // _CBLK)
    return pl.pallas_call(
        _argmin_body,
        grid=grid,
        in_specs=[
            pl.BlockSpec((_TBLK, _DIM), lambda t, c: (t, 0)),
            pl.BlockSpec((_CBLK, _DIM), lambda t, c: (c, 0)),
            pl.BlockSpec((_TBLK, 1), lambda t, c: (t, 0)),
            pl.BlockSpec((1, _CBLK), lambda t, c: (0, c)),
        ],
        out_specs=pl.BlockSpec((_TBLK, 1), lambda t, c: (t, 0)),
        out_shape=jax.ShapeDtypeStruct((_N_TOKENS, 1), jnp.int32),
        scratch_shapes=[
            pltpu.VMEM((_TBLK, 1), jnp.float32),
            pltpu.VMEM((_TBLK, 1), jnp.int32),
        ],
    )(x, cb, t1, t3)


def kernel(input_data, codebooks):
    remainder = input_data
    final_quantized = jnp.zeros_like(input_data)
    used = []
    for i in range(codebooks.shape[0]):
        cb = codebooks[i]
        t1 = jnp.sum(remainder ** 2, axis=1, keepdims=True)
        t3 = jnp.sum(cb.T ** 2, axis=0, keepdims=True)
        min_idx = _stage_argmin(remainder, cb, t1, t3)[:, 0]
        q = cb[min_idx]
        remainder = remainder - q
        final_quantized = final_quantized + q
        used.append(
            jnp.zeros((_N_CODES,), jnp.int32).at[min_idx].set(1))
    codebooks_used = jnp.stack(used, axis=0)
    return final_quantized, codebooks_used, codebooks


# speedup vs baseline: 1.1734x; 1.0019x over previous
import functools

import jax
import jax.numpy as jnp
from jax.experimental import pallas as pl
from jax.experimental.pallas import tpu as pltpu

_N_TOKENS = 8192
_N_CODES = 8192
_DIM = 256
_TBLK = 2048
_CBLK = 2048


def _argmin_body(x_ref, cb_ref, t1_ref, t3_ref, idx_ref, minval, minidx):
    c = pl.program_id(1)
    ncb = pl.num_programs(1)
    mm = jax.lax.dot_general(
        x_ref[...], cb_ref[...].astype(jnp.bfloat16),
        dimension_numbers=(((1,), (1,)), ((), ())),
        preferred_element_type=jnp.float32,
    )
    d = (t1_ref[...] - 2.0 * mm) + t3_ref[...]
    bmin = jnp.min(d, axis=1, keepdims=True)
    iota = jax.lax.broadcasted_iota(jnp.int32, d.shape, 1)
    big = jnp.int32(2**30)
    bidx = jnp.min(jnp.where(d == bmin, iota, big), axis=1, keepdims=True)
    bidx = bidx + c * _CBLK

    @pl.when(c == 0)
    def _():
        minval[...] = bmin
        minidx[...] = bidx

    @pl.when(c > 0)
    def _():
        better = bmin < minval[...]
        minval[...] = jnp.where(better, bmin, minval[...])
        minidx[...] = jnp.where(better, bidx, minidx[...])

    @pl.when(c == ncb - 1)
    def _():
        idx_ref[...] = minidx[...]


@functools.partial(jax.jit, static_argnums=())
def _stage_argmin(x, cb, t1, t3):
    grid = (_N_TOKENS // _TBLK, _N_CODES // _CBLK)
    return pl.pallas_call(
        _argmin_body,
        grid=grid,
        in_specs=[
            pl.BlockSpec((_TBLK, _DIM), lambda t, c: (t, 0)),
            pl.BlockSpec((_CBLK, _DIM), lambda t, c: (c, 0)),
            pl.BlockSpec((_TBLK, 1), lambda t, c: (t, 0)),
            pl.BlockSpec((1, _CBLK), lambda t, c: (0, c)),
        ],
        out_specs=pl.BlockSpec((_TBLK, 1), lambda t, c: (t, 0)),
        out_shape=jax.ShapeDtypeStruct((_N_TOKENS, 1), jnp.int32),
        scratch_shapes=[
            pltpu.VMEM((_TBLK, 1), jnp.float32),
            pltpu.VMEM((_TBLK, 1), jnp.int32),
        ],
    )(x, cb, t1, t3)


def kernel(input_data, codebooks):
    remainder = input_data
    final_quantized = jnp.zeros_like(input_data)
    used = []
    for i in range(codebooks.shape[0]):
        cb = codebooks[i]
        t1 = jnp.sum(remainder ** 2, axis=1, keepdims=True)
        t3 = jnp.sum(cb.T ** 2, axis=0, keepdims=True)
        min_idx = _stage_argmin(remainder, cb, t1, t3)[:, 0]
        q = cb[min_idx]
        remainder = remainder - q
        final_quantized = final_quantized + q
        used.append(
            jnp.zeros((_N_CODES,), jnp.int32).at[min_idx].set(1))
    codebooks_used = jnp.stack(used, axis=0)
    return final_quantized, codebooks_used, codebooks


# speedup vs baseline: 1.1744x; 1.0008x over previous
import functools

import jax
import jax.numpy as jnp
from jax.experimental import pallas as pl
from jax.experimental.pallas import tpu as pltpu

_N_TOKENS = 8192
_N_CODES = 8192
_DIM = 256
_TBLK = 2048
_CBLK = 2048


def _argmax_body(x_ref, cb_ref, t3h_ref, idx_ref, maxval, maxidx):
    c = pl.program_id(1)
    ncb = pl.num_programs(1)
    mm = jax.lax.dot_general(
        x_ref[...], cb_ref[...],
        dimension_numbers=(((1,), (1,)), ((), ())),
        preferred_element_type=jnp.float32,
    )
    s = mm - t3h_ref[...]
    bmax = jnp.max(s, axis=1, keepdims=True)
    iota = jax.lax.broadcasted_iota(jnp.int32, s.shape, 1)
    big = jnp.int32(2**30)
    bidx = jnp.min(jnp.where(s == bmax, iota, big), axis=1, keepdims=True)
    bidx = bidx + c * _CBLK

    @pl.when(c == 0)
    def _():
        maxval[...] = bmax
        maxidx[...] = bidx

    @pl.when(c > 0)
    def _():
        better = bmax > maxval[...]
        maxval[...] = jnp.where(better, bmax, maxval[...])
        maxidx[...] = jnp.where(better, bidx, maxidx[...])

    @pl.when(c == ncb - 1)
    def _():
        idx_ref[...] = maxidx[...]


def _stage_argmin(x, cb, t3h):
    grid = (_N_TOKENS // _TBLK, _N_CODES // _CBLK)
    return pl.pallas_call(
        _argmax_body,
        grid=grid,
        in_specs=[
            pl.BlockSpec((_TBLK, _DIM), lambda t, c: (t, 0)),
            pl.BlockSpec((_CBLK, _DIM), lambda t, c: (c, 0)),
            pl.BlockSpec((1, _CBLK), lambda t, c: (0, c)),
        ],
        out_specs=pl.BlockSpec((_TBLK, 1), lambda t, c: (t, 0)),
        out_shape=jax.ShapeDtypeStruct((_N_TOKENS, 1), jnp.int32),
        scratch_shapes=[
            pltpu.VMEM((_TBLK, 1), jnp.float32),
            pltpu.VMEM((_TBLK, 1), jnp.int32),
        ],
    )(x, cb, t3h)


def kernel(input_data, codebooks):
    remainder = input_data
    final_quantized = jnp.zeros_like(input_data)
    used = []
    for i in range(codebooks.shape[0]):
        cb = codebooks[i]
        t3h = 0.5 * jnp.sum(cb.T ** 2, axis=0, keepdims=True)
        min_idx = _stage_argmin(remainder, cb, t3h)[:, 0]
        q = cb[min_idx]
        remainder = remainder - q
        final_quantized = final_quantized + q
        used.append(
            jnp.zeros((_N_CODES,), jnp.int32).at[min_idx].set(1))
    codebooks_used = jnp.stack(used, axis=0)
    return final_quantized, codebooks_used, codebooks


# speedup vs baseline: 1.2038x; 1.0251x over previous
import functools

import jax
import jax.numpy as jnp
from jax.experimental import pallas as pl
from jax.experimental.pallas import tpu as pltpu

_N_TOKENS = 8192
_N_CODES = 8192
_DIM = 256
_TBLK = 2048
_CBLK = 4096


def _argmax_body(x_ref, cb_ref, t3h_ref, idx_ref, maxval, maxidx):
    c = pl.program_id(1)
    ncb = pl.num_programs(1)
    mm = jax.lax.dot_general(
        x_ref[...], cb_ref[...],
        dimension_numbers=(((1,), (1,)), ((), ())),
        preferred_element_type=jnp.float32,
    )
    s = mm - t3h_ref[...]
    bmax = jnp.max(s, axis=1, keepdims=True)
    iota = jax.lax.broadcasted_iota(jnp.int32, s.shape, 1)
    big = jnp.int32(2**30)
    bidx = jnp.min(jnp.where(s == bmax, iota, big), axis=1, keepdims=True)
    bidx = bidx + c * _CBLK

    @pl.when(c == 0)
    def _():
        maxval[...] = bmax
        maxidx[...] = bidx

    @pl.when(c > 0)
    def _():
        better = bmax > maxval[...]
        maxval[...] = jnp.where(better, bmax, maxval[...])
        maxidx[...] = jnp.where(better, bidx, maxidx[...])

    @pl.when(c == ncb - 1)
    def _():
        idx_ref[...] = maxidx[...]


def _stage_argmin(x, cb, t3h):
    grid = (_N_TOKENS // _TBLK, _N_CODES // _CBLK)
    return pl.pallas_call(
        _argmax_body,
        grid=grid,
        in_specs=[
            pl.BlockSpec((_TBLK, _DIM), lambda t, c: (t, 0)),
            pl.BlockSpec((_CBLK, _DIM), lambda t, c: (c, 0)),
            pl.BlockSpec((1, _CBLK), lambda t, c: (0, c)),
        ],
        out_specs=pl.BlockSpec((_TBLK, 1), lambda t, c: (t, 0)),
        out_shape=jax.ShapeDtypeStruct((_N_TOKENS, 1), jnp.int32),
        scratch_shapes=[
            pltpu.VMEM((_TBLK, 1), jnp.float32),
            pltpu.VMEM((_TBLK, 1), jnp.int32),
        ],
    )(x, cb, t3h)


def kernel(input_data, codebooks):
    remainder = input_data
    final_quantized = jnp.zeros_like(input_data)
    used = []
    for i in range(codebooks.shape[0]):
        cb = codebooks[i]
        t3h = 0.5 * jnp.sum(cb.T ** 2, axis=0, keepdims=True)
        min_idx = _stage_argmin(remainder, cb, t3h)[:, 0]
        q = cb[min_idx]
        remainder = remainder - q
        final_quantized = final_quantized + q
        used.append(
            jnp.zeros((_N_CODES,), jnp.int32).at[min_idx].set(1))
    codebooks_used = jnp.stack(used, axis=0)
    return final_quantized, codebooks_used, codebooks


# speedup vs baseline: 1.2103x; 1.0054x over previous
import functools

import jax
import jax.numpy as jnp
from jax.experimental import pallas as pl
from jax.experimental.pallas import tpu as pltpu

_N_TOKENS = 8192
_N_CODES = 8192
_DIM = 256
_TBLK = 1024
_CBLK = 8192


def _argmax_body(x_ref, cb_ref, t3h_ref, idx_ref, maxval, maxidx):
    c = pl.program_id(1)
    ncb = pl.num_programs(1)
    mm = jax.lax.dot_general(
        x_ref[...], cb_ref[...],
        dimension_numbers=(((1,), (1,)), ((), ())),
        preferred_element_type=jnp.float32,
    )
    s = mm - t3h_ref[...]
    bmax = jnp.max(s, axis=1, keepdims=True)
    iota = jax.lax.broadcasted_iota(jnp.int32, s.shape, 1)
    big = jnp.int32(2**30)
    bidx = jnp.min(jnp.where(s == bmax, iota, big), axis=1, keepdims=True)
    bidx = bidx + c * _CBLK

    @pl.when(c == 0)
    def _():
        maxval[...] = bmax
        maxidx[...] = bidx

    @pl.when(c > 0)
    def _():
        better = bmax > maxval[...]
        maxval[...] = jnp.where(better, bmax, maxval[...])
        maxidx[...] = jnp.where(better, bidx, maxidx[...])

    @pl.when(c == ncb - 1)
    def _():
        idx_ref[...] = maxidx[...]


def _stage_argmin(x, cb, t3h):
    grid = (_N_TOKENS // _TBLK, _N_CODES // _CBLK)
    return pl.pallas_call(
        _argmax_body,
        grid=grid,
        in_specs=[
            pl.BlockSpec((_TBLK, _DIM), lambda t, c: (t, 0)),
            pl.BlockSpec((_CBLK, _DIM), lambda t, c: (c, 0)),
            pl.BlockSpec((1, _CBLK), lambda t, c: (0, c)),
        ],
        out_specs=pl.BlockSpec((_TBLK, 1), lambda t, c: (t, 0)),
        out_shape=jax.ShapeDtypeStruct((_N_TOKENS, 1), jnp.int32),
        scratch_shapes=[
            pltpu.VMEM((_TBLK, 1), jnp.float32),
            pltpu.VMEM((_TBLK, 1), jnp.int32),
        ],
    )(x, cb, t3h)


def kernel(input_data, codebooks):
    remainder = input_data
    final_quantized = jnp.zeros_like(input_data)
    used = []
    for i in range(codebooks.shape[0]):
        cb = codebooks[i]
        t3h = 0.5 * jnp.sum(cb.T ** 2, axis=0, keepdims=True)
        min_idx = _stage_argmin(remainder, cb, t3h)[:, 0]
        q = cb[min_idx]
        remainder = remainder - q
        final_quantized = final_quantized + q
        used.append(
            jnp.zeros((_N_CODES,), jnp.int32).at[min_idx].set(1))
    codebooks_used = jnp.stack(used, axis=0)
    return final_quantized, codebooks_used, codebooks
